# Initial kernel scaffold; baseline (speedup 1.0000x reference)
import jax
import jax.numpy as jnp
from jax.experimental import pallas as pl


def kernel(input, target, idx_expanded):
    raise NotImplementedError("write your pallas kernel here")


# speedup vs baseline: 1.7450x; 1.7450x over previous
import functools

import jax
import jax.numpy as jnp
from jax import lax
from jax.experimental import pallas as pl
from jax.experimental.pallas import tpu as pltpu
from jax.experimental.pallas import tpu_sc as plsc

LANES = 16
NUM_WORKERS = 32


def kernel(input, target, idx_expanded):
    b, n, _, h, w = input.shape
    s = idx_expanded.shape[2]
    p_total = b * n
    pairs_per_worker = p_total // NUM_WORKERS

    inp = input.reshape(p_total, h, w)
    tgt = target.reshape(p_total, h, w)
    idx = idx_expanded.reshape(p_total, s, h)

    mesh = plsc.VectorSubcoreMesh(core_axis_name="c", subcore_axis_name="s")

    @functools.partial(
        pl.kernel,
        mesh=mesh,
        compiler_params=pltpu.CompilerParams(
            use_tc_tiling_on_sc=False, needs_layout_passes=False
        ),
        out_type=jax.ShapeDtypeStruct((NUM_WORKERS, LANES), jnp.float32),
        scratch_types=[
            pltpu.VMEM((h, s), jnp.float32),
            pltpu.VMEM((h, s), jnp.float32),
            pltpu.VMEM((s, h), jnp.int32),
            pltpu.VMEM((LANES,), jnp.float32),
            pltpu.SemaphoreType.DMA,
        ],
    )
    def sc_kernel(inp_hbm, tgt_hbm, idx_hbm, out_hbm, in_v, tg_v, ix_v, acc_v, sem):
        wid = lax.axis_index("s") * 2 + lax.axis_index("c")
        base = wid * pairs_per_worker

        def pair_body(pp, acc):
            p = base + pp
            pltpu.sync_copy(inp_hbm.at[p, :, pl.ds(0, s)], in_v)
            pltpu.sync_copy(tgt_hbm.at[p, :, pl.ds(0, s)], tg_v)
            pltpu.sync_copy(idx_hbm.at[p], ix_v)

            def row_body(i, acc_i):
                col = jnp.full((LANES,), i, dtype=jnp.int32)
                for jc in range(h // LANES):
                    iv = ix_v[i, pl.ds(jc * LANES, LANES)]
                    a = plsc.load_gather(in_v, [iv, col])
                    t = plsc.load_gather(tg_v, [iv, col])
                    d = a - t
                    acc_i = acc_i + d * d
                return acc_i

            return lax.fori_loop(0, s, row_body, acc)

        acc = lax.fori_loop(
            0, pairs_per_worker, pair_body, jnp.zeros((LANES,), jnp.float32)
        )
        acc_v[...] = acc
        pltpu.sync_copy(acc_v, out_hbm.at[wid])

    partials = sc_kernel(inp, tgt, idx)
    return jnp.sum(partials) / (b * n * s * h)


# speedup vs baseline: 3.1203x; 1.7882x over previous
import functools

import jax
import jax.numpy as jnp
from jax import lax
from jax.experimental import pallas as pl
from jax.experimental.pallas import tpu as pltpu
from jax.experimental.pallas import tpu_sc as plsc

LANES = 16
NUM_WORKERS = 32


def kernel(input, target, idx_expanded):
    b, n, _, h, w = input.shape
    s = idx_expanded.shape[2]
    p_total = b * n
    pairs_per_worker = p_total // NUM_WORKERS

    inp = input.reshape(p_total, h, w)
    tgt = target.reshape(p_total, h, w)
    idx = idx_expanded.reshape(p_total, s, h)

    mesh = plsc.VectorSubcoreMesh(core_axis_name="c", subcore_axis_name="s")

    @functools.partial(
        pl.kernel,
        mesh=mesh,
        compiler_params=pltpu.CompilerParams(
            use_tc_tiling_on_sc=False, needs_layout_passes=False
        ),
        out_type=jax.ShapeDtypeStruct((NUM_WORKERS, LANES), jnp.float32),
        scratch_types=[
            pltpu.VMEM((h, s + 1), jnp.float32),
            pltpu.VMEM((h, s + 1), jnp.float32),
            pltpu.VMEM((s, h), jnp.int32),
            pltpu.VMEM((LANES,), jnp.float32),
            pltpu.SemaphoreType.DMA,
        ],
    )
    def sc_kernel(inp_hbm, tgt_hbm, idx_hbm, out_hbm, in_v, tg_v, ix_v, acc_v, sem):
        wid = lax.axis_index("s") * 2 + lax.axis_index("c")
        base = wid * pairs_per_worker

        def pair_body(pp, acc):
            p = base + pp
            pltpu.sync_copy(inp_hbm.at[p, :, pl.ds(0, s)], in_v.at[:, pl.ds(0, s)])
            pltpu.sync_copy(tgt_hbm.at[p, :, pl.ds(0, s)], tg_v.at[:, pl.ds(0, s)])
            pltpu.sync_copy(idx_hbm.at[p], ix_v)

            def row_body(i, acc_i):
                col = jnp.full((LANES,), i, dtype=jnp.int32)
                for jc in range(h // LANES):
                    iv = ix_v[i, pl.ds(jc * LANES, LANES)]
                    a = plsc.load_gather(in_v, [iv, col])
                    t = plsc.load_gather(tg_v, [iv, col])
                    d = a - t
                    acc_i = acc_i + d * d
                return acc_i

            return lax.fori_loop(0, s, row_body, acc)

        acc = lax.fori_loop(
            0, pairs_per_worker, pair_body, jnp.zeros((LANES,), jnp.float32)
        )
        acc_v[...] = acc
        pltpu.sync_copy(acc_v, out_hbm.at[wid])

    partials = sc_kernel(inp, tgt, idx)
    return jnp.sum(partials) / (b * n * s * h)


# speedup vs baseline: 5.6027x; 1.7955x over previous
import functools

import jax
import jax.numpy as jnp
from jax import lax
from jax.experimental import pallas as pl
from jax.experimental.pallas import tpu as pltpu
from jax.experimental.pallas import tpu_sc as plsc

LANES = 16
NUM_WORKERS = 32


def kernel(input, target, idx_expanded):
    b, n, _, h, w = input.shape
    s = idx_expanded.shape[2]
    p_total = b * n
    pairs_per_worker = p_total // NUM_WORKERS

    inp = input.reshape(p_total, h, w)
    tgt = target.reshape(p_total, h, w)
    idx = idx_expanded.reshape(p_total, s, h)

    mesh = plsc.VectorSubcoreMesh(core_axis_name="c", subcore_axis_name="s")

    value_tile = pltpu.VMEM((h, s + 1), jnp.float32)
    index_tile = pltpu.VMEM((s, h), jnp.int32)

    @functools.partial(
        pl.kernel,
        mesh=mesh,
        compiler_params=pltpu.CompilerParams(
            use_tc_tiling_on_sc=False, needs_layout_passes=False
        ),
        out_type=jax.ShapeDtypeStruct((NUM_WORKERS, LANES), jnp.float32),
        scratch_types=[
            value_tile,
            value_tile,
            index_tile,
            value_tile,
            value_tile,
            index_tile,
            pltpu.VMEM((LANES,), jnp.float32),
            pltpu.SemaphoreType.DMA,
            pltpu.SemaphoreType.DMA,
        ],
    )
    def sc_kernel(
        inp_hbm, tgt_hbm, idx_hbm, out_hbm,
        in_a, tg_a, ix_a, in_b, tg_b, ix_b, acc_v, sem_a, sem_b,
    ):
        wid = lax.axis_index("s") * 2 + lax.axis_index("c")
        base = wid * pairs_per_worker
        bufs = ((in_a, tg_a, ix_a, sem_a), (in_b, tg_b, ix_b, sem_b))

        def copies(p, buf):
            in_v, tg_v, ix_v, sem = buf
            return (
                pltpu.make_async_copy(
                    inp_hbm.at[p, :, pl.ds(0, s)], in_v.at[:, pl.ds(0, s)], sem
                ),
                pltpu.make_async_copy(
                    tgt_hbm.at[p, :, pl.ds(0, s)], tg_v.at[:, pl.ds(0, s)], sem
                ),
                pltpu.make_async_copy(idx_hbm.at[p], ix_v, sem),
            )

        def start(p, buf):
            for c in copies(p, buf):
                c.start()

        def wait(p, buf):
            for c in copies(p, buf):
                c.wait()

        def compute(buf, acc):
            in_v, tg_v, ix_v, _ = buf

            @pl.loop(0, h)
            def _(q):
                for c in range(s // LANES):
                    sl = pl.ds(c * LANES, LANES)
                    in_v[q, sl] = in_v[q, sl] - tg_v[q, sl]

            def row_body(i, acc_i):
                col = jnp.full((LANES,), i, dtype=jnp.int32)
                for jc in range(h // LANES):
                    iv = ix_v[i, pl.ds(jc * LANES, LANES)]
                    d = plsc.load_gather(in_v, [iv, col])
                    acc_i = acc_i + d * d
                return acc_i

            return lax.fori_loop(0, s, row_body, acc)

        start(base, bufs[0])
        start(base + 1, bufs[1])

        def pair_pair_body(k, acc):
            pp = 2 * k
            p0 = base + pp
            p1 = base + pp + 1
            wait(p0, bufs[0])
            acc = compute(bufs[0], acc)

            @pl.when(pp + 2 < pairs_per_worker)
            def _():
                start(p0 + 2, bufs[0])

            wait(p1, bufs[1])
            acc = compute(bufs[1], acc)

            @pl.when(pp + 3 < pairs_per_worker)
            def _():
                start(p1 + 2, bufs[1])

            return acc

        acc = lax.fori_loop(
            0, pairs_per_worker // 2, pair_pair_body,
            jnp.zeros((LANES,), jnp.float32),
        )
        acc_v[...] = acc
        pltpu.sync_copy(acc_v, out_hbm.at[wid])

    partials = sc_kernel(inp, tgt, idx)
    return jnp.sum(partials) / (b * n * s * h)
